# baseline (speedup 1.0000x reference)
import functools

import jax
import jax.numpy as jnp
from jax import lax
from jax.experimental import pallas as pl
from jax.experimental.pallas import tpu as pltpu
from jax.experimental.pallas import tpu_sc as plsc

VOCAB = 1000
D = 1000
BATCH = 1024
SEQ = 50
VP = 1024
NC, NS = 2, 16
NW = NC * NS
NK = D // 8
KPW = 4
NM = BATCH // 16

_mesh = plsc.VectorSubcoreMesh(core_axis_name="c", subcore_axis_name="s")


@functools.partial(
    pl.kernel,
    mesh=_mesh,
    out_type=jax.ShapeDtypeStruct((SEQ * D, BATCH), jnp.float32),
    compiler_params=pltpu.CompilerParams(use_tc_tiling_on_sc=True,
                                         needs_layout_passes=False),
    scratch_types=[
        pltpu.VMEM((SEQ, BATCH), jnp.int32),
        pltpu.VMEM((8, VP), jnp.float32),
        pltpu.VMEM((8, BATCH), jnp.float32),
        pltpu.VMEM((8, BATCH), jnp.float32),
        pltpu.SemaphoreType.DMA,
        pltpu.SemaphoreType.DMA,
    ],
)
def _emb_tgather(xt_hbm, tablet_hbm, out_hbm, xt_v, ech, outb0, outb1,
                 osem0, osem1):
    wid = lax.axis_index("s") * NC + lax.axis_index("c")
    outbs = (outb0, outb1)
    osems = (osem0, osem1)

    pltpu.sync_copy(xt_hbm, xt_v)

    for kk in range(KPW):
        kf = wid + NW * kk

        @pl.when(kf < NK)
        def _():
            pltpu.sync_copy(tablet_hbm.at[pl.ds(8 * kf, 8)], ech)

            def sbody(jj, carry):
                for b2 in range(2):
                    s = 2 * jj + b2
                    dst_prev = out_hbm.at[pl.ds(8 * ((s - 2) * NK + kf), 8)]

                    @pl.when(jj > 0)
                    def _():
                        pltpu.make_async_copy(outbs[b2], dst_prev,
                                              osems[b2]).wait()

                    def mbody(m, c):
                        xv = xt_v[s, pl.ds(16 * m, 16)]
                        for vp in range(8):
                            sub = jnp.full((16,), vp, dtype=jnp.int32)
                            vals = plsc.load_gather(ech, [sub, xv])
                            outbs[b2][vp, pl.ds(16 * m, 16)] = vals
                        return c

                    lax.fori_loop(0, NM, mbody, 0)
                    pltpu.async_copy(
                        outbs[b2],
                        out_hbm.at[pl.ds(8 * (s * NK + kf), 8)], osems[b2])
                return carry

            lax.fori_loop(0, SEQ // 2, sbody, 0)

            for b2 in range(2):
                s = SEQ - 2 + b2
                pltpu.make_async_copy(
                    outbs[b2],
                    out_hbm.at[pl.ds(8 * (s * NK + kf), 8)],
                    osems[b2]).wait()


def kernel(x, emb):
    tablet = jnp.pad(emb.T, ((0, 0), (0, VP - VOCAB)))
    xt = x.T.astype(jnp.int32)
    out = _emb_tgather(xt, tablet)
    return out.reshape(SEQ, D, BATCH).transpose(2, 0, 1)


# speedup vs baseline: 5.6431x; 5.6431x over previous
import functools

import jax
import jax.numpy as jnp
from jax import lax
from jax.experimental import pallas as pl
from jax.experimental.pallas import tpu as pltpu
from jax.experimental.pallas import tpu_sc as plsc

VOCAB = 1000
D = 1000
BATCH = 1024
SEQ = 50
VP = 1024
NC, NS = 2, 16
NW = NC * NS
NK = D // 8
KPW = 4
NM = BATCH // 16

_mesh = plsc.VectorSubcoreMesh(core_axis_name="c", subcore_axis_name="s")


@functools.partial(
    pl.kernel,
    mesh=_mesh,
    out_type=jax.ShapeDtypeStruct((SEQ * D, BATCH), jnp.float32),
    compiler_params=pltpu.CompilerParams(use_tc_tiling_on_sc=True,
                                         needs_layout_passes=False),
    scratch_types=[
        pltpu.VMEM((SEQ, BATCH), jnp.int32),
        pltpu.VMEM((8, VP), jnp.float32),
        pltpu.VMEM((8, BATCH), jnp.float32),
        pltpu.VMEM((8, BATCH), jnp.float32),
        pltpu.SemaphoreType.DMA,
        pltpu.SemaphoreType.DMA,
    ],
)
def _emb_tgather(xt_hbm, tablet_hbm, out_hbm, xt_v, ech, outb0, outb1,
                 osem0, osem1):
    wid = lax.axis_index("s") * NC + lax.axis_index("c")
    outbs = (outb0, outb1)
    osems = (osem0, osem1)

    pltpu.sync_copy(xt_hbm, xt_v)

    for kk in range(KPW):
        kf = wid + NW * kk

        @pl.when(kf < NK)
        def _():
            pltpu.sync_copy(tablet_hbm.at[pl.ds(8 * kf, 8)], ech)

            def sbody(jj, carry):
                for b2 in range(2):
                    s = 2 * jj + b2
                    dst_prev = out_hbm.at[pl.ds(8 * ((s - 2) * NK + kf), 8)]

                    @pl.when(jj > 0)
                    def _():
                        pltpu.make_async_copy(outbs[b2], dst_prev,
                                              osems[b2]).wait()

                    @functools.partial(plsc.parallel_loop, 0, NM,
                                       unroll=4)
                    def _(m):
                        xv = xt_v[s, pl.ds(16 * m, 16)]
                        for vp in range(8):
                            sub = jnp.full((16,), vp, dtype=jnp.int32)
                            vals = plsc.load_gather(ech, [sub, xv])
                            outbs[b2][vp, pl.ds(16 * m, 16)] = vals
                    pltpu.async_copy(
                        outbs[b2],
                        out_hbm.at[pl.ds(8 * (s * NK + kf), 8)], osems[b2])
                return carry

            lax.fori_loop(0, SEQ // 2, sbody, 0)

            for b2 in range(2):
                s = SEQ - 2 + b2
                pltpu.make_async_copy(
                    outbs[b2],
                    out_hbm.at[pl.ds(8 * (s * NK + kf), 8)],
                    osems[b2]).wait()


def kernel(x, emb):
    tablet = jnp.pad(emb.T, ((0, 0), (0, VP - VOCAB)))
    xt = x.T.astype(jnp.int32)
    out = _emb_tgather(xt, tablet)
    return out.reshape(SEQ, D, BATCH).transpose(2, 0, 1)
